# baseline (speedup 1.0000x reference)
import functools

import jax
import jax.numpy as jnp
from jax import lax
from jax.experimental import pallas as pl
from jax.experimental.pallas import tpu as pltpu
from jax.experimental.pallas import tpu_sc as plsc

_NC = 2
_NS = 16
_NW = _NC * _NS
_B = 16384
_D = 128
_BPW = _B // _NW
_CHUNK = 128
_NCHUNK = _BPW // _CHUNK
_VPR = _D // 16
_T = 1024
_TPT = _T // _NS


@functools.partial(
    pl.kernel,
    out_type=(
        jax.ShapeDtypeStruct((_NC, _T, _D), jnp.float32),
        jax.ShapeDtypeStruct((_NC, _T, _D), jnp.float32),
        jax.ShapeDtypeStruct((_NW, 16), jnp.float32),
    ),
    mesh=plsc.VectorSubcoreMesh(core_axis_name="c", subcore_axis_name="s"),
    scratch_types=[
        pltpu.VMEM((_NCHUNK, _CHUNK), jnp.int32),
        pltpu.VMEM((2, _CHUNK, _D), jnp.float32),
        pltpu.VMEM((_CHUNK, _D), jnp.float32),
        pltpu.VMEM((_TPT, _D), jnp.float32),
        pltpu.VMEM((16,), jnp.float32),
        pltpu.VMEM_SHARED((_T, _D), jnp.float32),
        pltpu.VMEM_SHARED((_T, _D), jnp.float32),
        pltpu.SemaphoreType.DMA,
        pltpu.SemaphoreType.DMA,
        pltpu.SemaphoreType.DMA,
        pltpu.SemaphoreType.DMA,
    ],
)
def _sc_segment_sums(feat_hbm, lab_hbm, s_hbm, cnt_hbm, f2_hbm,
                     idx_v, fbuf, ones_v, zrow_v, acc_v,
                     s_sp, cnt_sp, sf0, sf1, ss0, ss1):
    cid = lax.axis_index("c")
    sid = lax.axis_index("s")
    wid = sid * _NC + cid
    base = wid * _BPW

    pltpu.sync_copy(lab_hbm.at[pl.ds(wid * _NCHUNK, _NCHUNK)], idx_v)

    def fill_body(i, _):
        for k in range(_VPR):
            ones_v[i, pl.ds(k * 16, 16)] = jnp.full((16,), 1.0, jnp.float32)
        return 0
    lax.fori_loop(0, _CHUNK, fill_body, 0)

    def zero_body(i, _):
        for k in range(_VPR):
            zrow_v[i, pl.ds(k * 16, 16)] = jnp.zeros((16,), jnp.float32)
        return 0
    lax.fori_loop(0, _TPT, zero_body, 0)

    pltpu.sync_copy(zrow_v, s_sp.at[pl.ds(sid * _TPT, _TPT)])
    pltpu.sync_copy(zrow_v, cnt_sp.at[pl.ds(sid * _TPT, _TPT)])
    plsc.subcore_barrier()

    fsems = (sf0, sf1)
    ssems = (ss0, ss1)

    def start_feat(j):
        slot = j % 2
        return pltpu.async_copy(
            feat_hbm.at[pl.ds(base + j * _CHUNK, _CHUNK)],
            fbuf.at[slot], fsems[slot])

    pending_f = start_feat(0)
    accs = tuple(jnp.zeros((16,), jnp.float32) for _ in range(_VPR))
    for j in range(_NCHUNK):
        slot = j % 2
        pending_f.wait()
        sd = pltpu.async_copy(fbuf.at[slot], s_sp.at[idx_v.at[j]],
                              ssems[slot], add=True)
        cd = pltpu.async_copy(ones_v, cnt_sp.at[idx_v.at[j]],
                              ssems[slot], add=True)

        def row_body(i, a):
            a = list(a)
            for r in range(2):
                row = i * 2 + r
                for k in range(_VPR):
                    f = fbuf[slot, row, pl.ds(k * 16, 16)]
                    a[k] = a[k] + f * f
            return tuple(a)
        accs = lax.fori_loop(0, _CHUNK // 2, row_body, accs)

        sd.wait()
        cd.wait()
        if j + 1 < _NCHUNK:
            pending_f = start_feat(j + 1)

    acc = accs[0]
    for k in range(1, _VPR):
        acc = acc + accs[k]
    acc_v[...] = acc
    pltpu.sync_copy(acc_v, f2_hbm.at[wid])

    plsc.subcore_barrier()
    pltpu.sync_copy(s_sp.at[pl.ds(sid * _TPT, _TPT)], zrow_v)
    pltpu.sync_copy(zrow_v, s_hbm.at[cid, pl.ds(sid * _TPT, _TPT)])
    pltpu.sync_copy(cnt_sp.at[pl.ds(sid * _TPT, _TPT)], zrow_v)
    pltpu.sync_copy(zrow_v, cnt_hbm.at[cid, pl.ds(sid * _TPT, _TPT)])


def _finish(s_ref, cnt_ref, c_ref, f2_ref, o_ref):
    s = s_ref[0] + s_ref[1]
    c = c_ref[...]
    n = cnt_ref[0, :, 0] + cnt_ref[1, :, 0]
    sc_dot = jnp.sum(s * c)
    cc = jnp.sum(n * jnp.sum(c * c, axis=1))
    f2 = jnp.sum(f2_ref[...])
    o_ref[0, 0] = (f2 - 2.0 * sc_dot + cc) * (1.0 / _B)


_finish_call = pl.pallas_call(
    _finish,
    out_shape=jax.ShapeDtypeStruct((1, 1), jnp.float32),
    out_specs=pl.BlockSpec(memory_space=pltpu.SMEM),
)


def kernel(features, labels, centers):
    labels2d = labels.astype(jnp.int32).reshape(_B // _CHUNK, _CHUNK)
    c_pad = jnp.zeros((_T, _D), jnp.float32).at[:centers.shape[0]].set(centers)
    s, cnt, f2 = _sc_segment_sums(features, labels2d)
    return _finish_call(s, cnt, c_pad, f2)[0, 0]


# speedup vs baseline: 1.2262x; 1.2262x over previous
import functools

import jax
import jax.numpy as jnp
from jax import lax
from jax.experimental import pallas as pl
from jax.experimental.pallas import tpu as pltpu
from jax.experimental.pallas import tpu_sc as plsc

_NC = 2
_NS = 16
_NW = _NC * _NS
_B = 16384
_D = 128
_DP = _D // 2
_BPW = _B // _NW
_CHUNK = 128
_NCHUNK = _BPW // _CHUNK
_GPR = _D // 32
_NBUF = 3


@functools.partial(
    pl.kernel,
    out_type=jax.ShapeDtypeStruct((_NW, 16), jnp.float32),
    mesh=plsc.VectorSubcoreMesh(core_axis_name="c", subcore_axis_name="s"),
    compiler_params=pltpu.CompilerParams(use_tc_tiling_on_sc=False),
    scratch_types=[
        pltpu.VMEM((_NCHUNK, _CHUNK), jnp.int32),
        pltpu.VMEM((_NBUF, _CHUNK, _D), jnp.float32),
        pltpu.VMEM((_NBUF, _CHUNK, _DP), jnp.float32),
        pltpu.VMEM((16,), jnp.float32),
        pltpu.SemaphoreType.DMA,
        pltpu.SemaphoreType.DMA,
        pltpu.SemaphoreType.DMA,
        pltpu.SemaphoreType.DMA,
        pltpu.SemaphoreType.DMA,
        pltpu.SemaphoreType.DMA,
    ],
)
def _sc_partials(feat_hbm, lab_hbm, cent_hbm, out_hbm,
                 idx_v, fbuf, cbuf, acc_v, sf0, sf1, sf2, sc0, sc1, sc2):
    wid = lax.axis_index("s") * _NC + lax.axis_index("c")
    base = wid * _BPW

    pltpu.sync_copy(lab_hbm.at[pl.ds(wid * _NCHUNK, _NCHUNK)], idx_v)

    fsems = (sf0, sf1, sf2)
    csems = (sc0, sc1, sc2)

    def start(j):
        slot = j % _NBUF
        fd = pltpu.async_copy(
            feat_hbm.at[pl.ds(base + j * _CHUNK, _CHUNK)],
            fbuf.at[slot], fsems[slot])
        cd = pltpu.async_copy(
            cent_hbm.at[idx_v.at[j]], cbuf.at[slot], csems[slot])
        return fd, cd

    pending = [start(0), start(1), start(2)]
    accs = [jnp.zeros((16,), jnp.float32) for _ in range(8)]
    for j in range(_NCHUNK):
        slot = j % _NBUF
        fd, cd = pending.pop(0)
        fd.wait()
        cd.wait()
        if j + _NBUF < _NCHUNK:
            pending.append(start(j + _NBUF))

        def row_body(i, a):
            a = list(a)
            for g in range(_GPR):
                w = cbuf[slot, i, pl.ds(g * 16, 16)]
                wi = lax.bitcast_convert_type(w, jnp.int32)
                ca = lax.bitcast_convert_type(wi << 16, jnp.float32)
                cb = lax.bitcast_convert_type(wi & jnp.int32(-65536), jnp.float32)
                fa = fbuf[slot, i, pl.ds(g * 32, 16)]
                fb = fbuf[slot, i, pl.ds(g * 32 + 16, 16)]
                da = fa - ca
                db = fb - cb
                a[2 * g] = a[2 * g] + da * da
                a[2 * g + 1] = a[2 * g + 1] + db * db
            return tuple(a)

        accs = list(lax.fori_loop(0, _CHUNK, row_body, tuple(accs)))

    acc = accs[0]
    for k in range(1, 8):
        acc = acc + accs[k]
    acc_v[...] = acc
    pltpu.sync_copy(acc_v, out_hbm.at[wid])


def _finish(p_ref, o_ref):
    o_ref[0, 0] = jnp.sum(p_ref[...]) * (1.0 / _B)


_finish_call = pl.pallas_call(
    _finish,
    out_shape=jax.ShapeDtypeStruct((1, 1), jnp.float32),
    out_specs=pl.BlockSpec(memory_space=pltpu.SMEM),
)


def _pack_centers(centers):
    cb = centers.astype(jnp.bfloat16).reshape(-1, _GPR, 2, 16)
    cb = cb.transpose(0, 1, 3, 2)
    cw = jax.lax.bitcast_convert_type(cb, jnp.float32)
    return cw.reshape(-1, _DP)


def kernel(features, labels, centers):
    labels2d = labels.astype(jnp.int32).reshape(_B // _CHUNK, _CHUNK)
    partials = _sc_partials(features, labels2d, _pack_centers(centers))
    return _finish_call(partials)[0, 0]


# speedup vs baseline: 1.2292x; 1.0024x over previous
import functools

import jax
import jax.numpy as jnp
from jax import lax
from jax.experimental import pallas as pl
from jax.experimental.pallas import tpu as pltpu
from jax.experimental.pallas import tpu_sc as plsc

_NC = 2
_NS = 16
_NW = _NC * _NS
_B = 16384
_D = 128
_DP = _D // 2
_BPW = _B // _NW
_CHUNK = 128
_NCHUNK = _BPW // _CHUNK
_GPR = _D // 32
_NBUF = 3


@functools.partial(
    pl.kernel,
    out_type=jax.ShapeDtypeStruct((_NW, 16), jnp.float32),
    mesh=plsc.VectorSubcoreMesh(core_axis_name="c", subcore_axis_name="s"),
    compiler_params=pltpu.CompilerParams(use_tc_tiling_on_sc=False),
    scratch_types=[
        pltpu.VMEM((_NCHUNK, _CHUNK), jnp.int32),
        pltpu.VMEM((_NBUF, _CHUNK, _D), jnp.float32),
        pltpu.VMEM((_NBUF, _CHUNK, _DP), jnp.float32),
        pltpu.VMEM((16,), jnp.float32),
        pltpu.SemaphoreType.DMA,
        pltpu.SemaphoreType.DMA,
        pltpu.SemaphoreType.DMA,
        pltpu.SemaphoreType.DMA,
        pltpu.SemaphoreType.DMA,
        pltpu.SemaphoreType.DMA,
    ],
)
def _sc_partials(feat_hbm, lab_hbm, cent_hbm, out_hbm,
                 idx_v, fbuf, cbuf, acc_v, sf0, sf1, sf2, sc0, sc1, sc2):
    wid = lax.axis_index("s") * _NC + lax.axis_index("c")
    base = wid * _BPW

    pltpu.sync_copy(lab_hbm.at[pl.ds(wid * _NCHUNK, _NCHUNK)], idx_v)

    fsems = (sf0, sf1, sf2)
    csems = (sc0, sc1, sc2)

    def start(j):
        slot = j % _NBUF
        fd = pltpu.async_copy(
            feat_hbm.at[pl.ds(base + j * _CHUNK, _CHUNK)],
            fbuf.at[slot], fsems[slot])
        cd = pltpu.async_copy(
            cent_hbm.at[idx_v.at[j]], cbuf.at[slot], csems[slot])
        return fd, cd

    pending = [start(0), start(1), start(2)]
    accs = [jnp.zeros((16,), jnp.float32) for _ in range(8)]
    for j in range(_NCHUNK):
        slot = j % _NBUF
        fd, cd = pending.pop(0)
        fd.wait()
        cd.wait()
        if j + _NBUF < _NCHUNK:
            pending.append(start(j + _NBUF))

        def row_body(i, a):
            a = list(a)
            for r in range(2):
                row = i * 2 + r
                for g in range(_GPR):
                    w = cbuf[slot, row, pl.ds(g * 16, 16)]
                    wi = lax.bitcast_convert_type(w, jnp.int32)
                    ca = lax.bitcast_convert_type(wi << 16, jnp.float32)
                    cb = lax.bitcast_convert_type(wi & jnp.int32(-65536),
                                                  jnp.float32)
                    fa = fbuf[slot, row, pl.ds(g * 32, 16)]
                    fb = fbuf[slot, row, pl.ds(g * 32 + 16, 16)]
                    da = fa - ca
                    db = fb - cb
                    a[2 * g] = a[2 * g] + da * da
                    a[2 * g + 1] = a[2 * g + 1] + db * db
            return tuple(a)

        accs = list(lax.fori_loop(0, _CHUNK // 2, row_body, tuple(accs)))

    acc = accs[0]
    for k in range(1, 8):
        acc = acc + accs[k]
    acc_v[...] = acc
    pltpu.sync_copy(acc_v, out_hbm.at[wid])


def _finish(p_ref, o_ref):
    o_ref[0, 0] = jnp.sum(p_ref[...]) * (1.0 / _B)


_finish_call = pl.pallas_call(
    _finish,
    out_shape=jax.ShapeDtypeStruct((1, 1), jnp.float32),
    out_specs=pl.BlockSpec(memory_space=pltpu.SMEM),
)


def _pack_centers(centers):
    cb = centers.astype(jnp.bfloat16).reshape(-1, _GPR, 2, 16)
    cb = cb.transpose(0, 1, 3, 2)
    cw = jax.lax.bitcast_convert_type(cb, jnp.float32)
    return cw.reshape(-1, _DP)


def kernel(features, labels, centers):
    labels2d = labels.astype(jnp.int32).reshape(_B // _CHUNK, _CHUNK)
    partials = _sc_partials(features, labels2d, _pack_centers(centers))
    return _finish_call(partials)[0, 0]
